# baseline (speedup 1.0000x reference)
import functools

import jax
import jax.numpy as jnp
from jax import lax
from jax.experimental import pallas as pl
from jax.experimental.pallas import tpu as pltpu
from jax.experimental.pallas import tpu_sc as plsc

B = 16
S = 512
KK = 64
N = 64
T = 64
D = N * T
NW = 32
ROWS_PER_W = (KK * B) // NW
CHUNK = 8
NCHUNK = ROWS_PER_W // CHUNK
A_WORDS_PER_W = ROWS_PER_W * S


@functools.partial(
    pl.kernel,
    out_type=(
        jax.ShapeDtypeStruct((KK * B, D), jnp.float32),
        jax.ShapeDtypeStruct((B * KK * S,), jnp.float32),
    ),
    mesh=plsc.VectorSubcoreMesh(core_axis_name="c", subcore_axis_name="s"),
    compiler_params=pltpu.CompilerParams(needs_layout_passes=False),
    scratch_types=[
        pltpu.VMEM((B * KK,), jnp.int32),
        pltpu.VMEM((ROWS_PER_W,), jnp.int32),
        pltpu.VMEM((CHUNK, D), jnp.float32),
        pltpu.VMEM((CHUNK, D), jnp.float32),
        pltpu.VMEM((A_WORDS_PER_W,), jnp.float32),
        pltpu.SemaphoreType.DMA,
        pltpu.SemaphoreType.DMA,
        pltpu.SemaphoreType.DMA,
        pltpu.SemaphoreType.DMA,
        pltpu.SemaphoreType.DMA,
    ],
)
def _sc_gather(y_hbm, idx_hbm, ysel_hbm, a_hbm,
               idx_v, gidx_v, buf0, buf1, a_v,
               gs0, gs1, os0, os1, asem):
    w = lax.axis_index("s") * 2 + lax.axis_index("c")
    lane = lax.broadcasted_iota(jnp.int32, (16,), 0)

    pltpu.sync_copy(idx_hbm, idx_v)

    for j in range(2):
        k = 2 * w + j
        vals = plsc.load_gather(idx_v, [lane * KK + k])
        gidx_v[pl.ds(j * 16, 16)] = lane * S + vals

    base = w * ROWS_PER_W

    def gather(ch, buf, sem):
        return pltpu.async_copy(
            y_hbm.at[gidx_v.at[pl.ds(ch * CHUNK, CHUNK)]], buf, sem)

    def put(ch, buf, sem):
        return pltpu.async_copy(
            buf, ysel_hbm.at[pl.ds(base + ch * CHUNK, CHUNK)], sem)

    g0 = gather(0, buf0, gs0)
    g1 = gather(1, buf1, gs1)

    zeros = jnp.zeros((16,), jnp.float32)

    def zero_body(i, carry):
        for c in range(ROWS_PER_W):
            a_v[pl.ds(i * S + c * 16, 16)] = zeros
        return carry

    lax.fori_loop(0, ROWS_PER_W, zero_body, 0)

    b = w // 2
    koff = (w % 2) * ROWS_PER_W
    ones = jnp.full((16,), 1.0, jnp.float32)
    for j in range(2):
        cols = idx_v[pl.ds(b * KK + koff + j * 16, 16)]
        offs = (j * 16 + lane) * S + cols
        plsc.store_scatter(a_v, [offs], ones)
    a_cp = pltpu.async_copy(
        a_v, a_hbm.at[pl.ds(w * A_WORDS_PER_W, A_WORDS_PER_W)], asem)

    g0.wait()
    o0 = put(0, buf0, os0)
    g1.wait()
    o1 = put(1, buf1, os1)
    o0.wait()
    g2 = gather(2, buf0, gs0)
    o1.wait()
    g3 = gather(3, buf1, gs1)
    g2.wait()
    o2 = put(2, buf0, os0)
    g3.wait()
    o3 = put(3, buf1, os1)
    o2.wait()
    o3.wait()
    a_cp.wait()


def kernel(Y_full, idx_all):
    y2 = Y_full.reshape(B * S, D)
    idx_flat = idx_all.reshape(-1)
    ysel_flat, a_flat = _sc_gather(y2, idx_flat)
    Y_sel = ysel_flat.reshape(KK, B, N, T)
    A = a_flat.reshape(B, KK, S)
    return (Y_sel, A, A)


# speedup vs baseline: 2.2623x; 2.2623x over previous
import functools

import jax
import jax.numpy as jnp
from jax import lax
from jax.experimental import pallas as pl
from jax.experimental.pallas import tpu as pltpu
from jax.experimental.pallas import tpu_sc as plsc

B = 16
S = 512
KK = 64
N = 64
T = 64
D = N * T
M = B * N * T
NW = 32
CM = 64
NCH = (D // 2) // CM
A_ROWS_PER_W = (B * KK) // NW
A_WORDS_PER_W = A_ROWS_PER_W * S


@functools.partial(
    pl.kernel,
    out_type=(
        jax.ShapeDtypeStruct((M // 2, 128), jnp.float32),
        jax.ShapeDtypeStruct((B * KK * S,), jnp.float32),
        jax.ShapeDtypeStruct((B * KK * S,), jnp.float32),
    ),
    mesh=plsc.VectorSubcoreMesh(core_axis_name="c", subcore_axis_name="s"),
    compiler_params=pltpu.CompilerParams(needs_layout_passes=False),
    scratch_types=[
        pltpu.VMEM((B * KK,), jnp.int32),
        pltpu.VMEM((CM, S), jnp.float32),
        pltpu.VMEM((CM, S), jnp.float32),
        pltpu.VMEM((KK, 128), jnp.float32),
        pltpu.VMEM((KK, 128), jnp.float32),
        pltpu.VMEM((KK,), jnp.int32),
        pltpu.VMEM((KK,), jnp.int32),
        pltpu.VMEM((A_WORDS_PER_W,), jnp.float32),
        pltpu.SemaphoreType.DMA,
        pltpu.SemaphoreType.DMA,
        pltpu.SemaphoreType.DMA,
        pltpu.SemaphoreType.DMA,
        pltpu.SemaphoreType.DMA,
        pltpu.SemaphoreType.DMA,
    ],
)
def _sc_filter(y_hbm, idx_hbm, ysel_hbm, a0_hbm, a1_hbm,
               idx_v, in0, in1, out0, out1, rid0, rid1, a_v,
               gs0, gs1, os0, os1, as0, as1):
    w = lax.axis_index("s") * 2 + lax.axis_index("c")
    b = w // 2
    half = w % 2
    lane = lax.broadcasted_iota(jnp.int32, (16,), 0)
    zero16 = jnp.zeros((16,), jnp.int32)

    pltpu.sync_copy(idx_hbm, idx_v)

    mbase = b * D + half * (D // 2)
    inb = (in0, in1)
    outb = (out0, out1)
    ridb = (rid0, rid1)
    gsem = (gs0, gs1)
    osem = (os0, os1)

    def copy_in(ch, p):
        return pltpu.async_copy(
            y_hbm.at[pl.ds(mbase + ch * CM, CM)], inb[p], gsem[p])

    gin = [copy_in(0, 0), copy_in(1, 1)]

    kidx = []
    kvec = []
    for j in range(4):
        kidx.append(plsc.load_gather(idx_v, [b * KK + j * 16 + lane]))
        kvec.append(j * 16 + lane)

    zf = jnp.zeros((16,), jnp.float32)

    def zero_body(i, carry):
        for c in range(32):
            a_v[pl.ds(i * S + c * 16, 16)] = zf
        return carry

    lax.fori_loop(0, A_ROWS_PER_W, zero_body, 0)

    ab = w // 2
    koff = (w % 2) * A_ROWS_PER_W
    ones = jnp.full((16,), 1.0, jnp.float32)
    for j in range(2):
        cols = idx_v[pl.ds(ab * KK + koff + j * 16, 16)]
        plsc.store_scatter(a_v, [(j * 16 + lane) * S + cols], ones)
    a_cp0 = pltpu.async_copy(
        a_v, a0_hbm.at[pl.ds(w * A_WORDS_PER_W, A_WORDS_PER_W)], as0)
    a_cp1 = pltpu.async_copy(
        a_v, a1_hbm.at[pl.ds(w * A_WORDS_PER_W, A_WORDS_PER_W)], as1)

    rbase = b * 32 + half * (NCH // 2)

    gout = [None, None]
    for ch in range(NCH):
        p = ch % 2
        q = ch // 2
        qp = q % 2
        if ch % 2 == 0 and gout[qp] is not None:
            gout[qp].wait()
        gin[p].wait()

        src = inb[p]
        dst = outb[qp]
        coff = (ch % 2) * CM

        def body(m, carry, src=src, dst=dst, coff=coff):
            mv = zero16 + m
            for j in range(4):
                v = plsc.load_gather(src, [mv, kidx[j]])
                plsc.store_scatter(dst, [kvec[j], mv + coff], v)
            return carry

        lax.fori_loop(0, CM, body, 0)

        if ch % 2 == 1:
            rv = q + rbase
            for j in range(4):
                ridb[qp][pl.ds(j * 16, 16)] = kvec[j] * 512 + rv
            gout[qp] = pltpu.async_copy(
                outb[qp], ysel_hbm.at[ridb[qp]], osem[qp])
        if ch + 2 < NCH:
            gin[p] = copy_in(ch + 2, p)

    gout[0].wait()
    gout[1].wait()
    a_cp0.wait()
    a_cp1.wait()


def kernel(Y_full, idx_all):
    y_t = jnp.transpose(Y_full, (0, 2, 3, 1)).reshape(M, S)
    idx_flat = idx_all.reshape(-1)
    ysel_fine, a0, a1 = _sc_filter(y_t, idx_flat)
    Y_sel = ysel_fine.reshape(KK, B, N, T)
    A = a0.reshape(B, KK, S)
    A2 = a1.reshape(B, KK, S)
    return (Y_sel, A, A2)


# speedup vs baseline: 2.6630x; 1.1771x over previous
import functools

import jax
import jax.numpy as jnp
from jax import lax
from jax.experimental import pallas as pl
from jax.experimental.pallas import tpu as pltpu
from jax.experimental.pallas import tpu_sc as plsc

B = 16
S = 512
KK = 64
N = 64
T = 64
D = N * T
M = B * N * T
NW = 32
CM = 64
NCH = (D // 2) // CM
A_ROWS_PER_W = (B * KK) // NW
A_WORDS_PER_W = A_ROWS_PER_W * S


@functools.partial(
    pl.kernel,
    out_type=(
        jax.ShapeDtypeStruct((M // 2, 128), jnp.float32),
        jax.ShapeDtypeStruct((B * KK * S,), jnp.float32),
        jax.ShapeDtypeStruct((B * KK * S,), jnp.float32),
    ),
    mesh=plsc.VectorSubcoreMesh(core_axis_name="c", subcore_axis_name="s"),
    compiler_params=pltpu.CompilerParams(needs_layout_passes=False),
    scratch_types=[
        pltpu.VMEM((B * KK,), jnp.int32),
        pltpu.VMEM((CM, S), jnp.float32),
        pltpu.VMEM((CM, S), jnp.float32),
        pltpu.VMEM((KK, 128), jnp.float32),
        pltpu.VMEM((KK, 128), jnp.float32),
        pltpu.VMEM((KK,), jnp.int32),
        pltpu.VMEM((KK,), jnp.int32),
        pltpu.VMEM((A_WORDS_PER_W,), jnp.float32),
        pltpu.SemaphoreType.DMA,
        pltpu.SemaphoreType.DMA,
        pltpu.SemaphoreType.DMA,
        pltpu.SemaphoreType.DMA,
        pltpu.SemaphoreType.DMA,
        pltpu.SemaphoreType.DMA,
    ],
)
def _sc_filter(y_hbm, idx_hbm, ysel_hbm, a0_hbm, a1_hbm,
               idx_v, in0, in1, out0, out1, rid0, rid1, a_v,
               gs0, gs1, os0, os1, as0, as1):
    w = lax.axis_index("s") * 2 + lax.axis_index("c")
    b = w // 2
    half = w % 2
    lane = lax.broadcasted_iota(jnp.int32, (16,), 0)
    zero16 = jnp.zeros((16,), jnp.int32)

    pltpu.sync_copy(idx_hbm, idx_v)

    mbase = b * D + half * (D // 2)
    inb = (in0, in1)
    outb = (out0, out1)
    ridb = (rid0, rid1)
    gsem = (gs0, gs1)
    osem = (os0, os1)

    def copy_in(ch, p):
        return pltpu.async_copy(
            y_hbm.at[pl.ds(mbase + ch * CM, CM)], inb[p], gsem[p])

    gin = [copy_in(0, 0), copy_in(1, 1)]

    kidx = []
    kvec = []
    for j in range(4):
        kidx.append(plsc.load_gather(idx_v, [b * KK + j * 16 + lane]))
        kvec.append(j * 16 + lane)

    zf = jnp.zeros((16,), jnp.float32)

    @plsc.parallel_loop(0, A_ROWS_PER_W, 1, unroll=2)
    def _zero(i):
        for c in range(32):
            a_v[pl.ds(i * S + c * 16, 16)] = zf

    ab = w // 2
    koff = (w % 2) * A_ROWS_PER_W
    ones = jnp.full((16,), 1.0, jnp.float32)
    for j in range(2):
        cols = idx_v[pl.ds(ab * KK + koff + j * 16, 16)]
        plsc.store_scatter(a_v, [(j * 16 + lane) * S + cols], ones)
    a_cp0 = pltpu.async_copy(
        a_v, a0_hbm.at[pl.ds(w * A_WORDS_PER_W, A_WORDS_PER_W)], as0)
    a_cp1 = pltpu.async_copy(
        a_v, a1_hbm.at[pl.ds(w * A_WORDS_PER_W, A_WORDS_PER_W)], as1)

    rbase = b * 32 + half * (NCH // 2)

    gout = [None, None]
    for ch in range(NCH):
        p = ch % 2
        q = ch // 2
        qp = q % 2
        if ch % 2 == 0 and gout[qp] is not None:
            gout[qp].wait()
        gin[p].wait()

        src = inb[p]
        dst = outb[qp]
        coff = (ch % 2) * CM

        @plsc.parallel_loop(0, CM, 1, unroll=8)
        def _extract(m, src=src, dst=dst, coff=coff):
            mv = zero16 + m
            for j in range(4):
                v = plsc.load_gather(src, [mv, kidx[j]])
                plsc.store_scatter(dst, [kvec[j], mv + coff], v)

        if ch % 2 == 1:
            rv = q + rbase
            for j in range(4):
                ridb[qp][pl.ds(j * 16, 16)] = kvec[j] * 512 + rv
            gout[qp] = pltpu.async_copy(
                outb[qp], ysel_hbm.at[ridb[qp]], osem[qp])
        if ch + 2 < NCH:
            gin[p] = copy_in(ch + 2, p)

    gout[0].wait()
    gout[1].wait()
    a_cp0.wait()
    a_cp1.wait()


def kernel(Y_full, idx_all):
    y_t = jnp.transpose(Y_full, (0, 2, 3, 1)).reshape(M, S)
    idx_flat = idx_all.reshape(-1)
    ysel_fine, a0, a1 = _sc_filter(y_t, idx_flat)
    Y_sel = ysel_fine.reshape(KK, B, N, T)
    A = a0.reshape(B, KK, S)
    A2 = a1.reshape(B, KK, S)
    return (Y_sel, A, A2)
